# Initial kernel scaffold; baseline (speedup 1.0000x reference)
import jax
import jax.numpy as jnp
from jax.experimental import pallas as pl


def kernel(boxes, scores, labels):
    raise NotImplementedError("write your pallas kernel here")


# speedup vs baseline: 240.7936x; 240.7936x over previous
import functools

import jax
import jax.numpy as jnp
from jax import lax
from jax.experimental import pallas as pl
from jax.experimental.pallas import tpu as pltpu
from jax.experimental.pallas import tpu_sc as plsc

_B = 4
_N = 5000
_L = 16
_NP = 5024
_NCH2 = _NP // (2 * _L)
_MAXOUT = 300
_IOU_THR = 0.6
_SCORE_THR = 0.01
_NEG_INF = float("-inf")

_mesh = plsc.VectorSubcoreMesh(core_axis_name="c", subcore_axis_name="s")


@functools.partial(
    pl.kernel,
    out_type=jax.ShapeDtypeStruct((_B, _MAXOUT * 8), jnp.float32),
    mesh=_mesh,
    compiler_params=pltpu.CompilerParams(needs_layout_passes=False),
    scratch_types=[
        pltpu.VMEM((_NP,), jnp.float32),
        pltpu.VMEM((_NP,), jnp.float32),
        pltpu.VMEM((_NP,), jnp.float32),
        pltpu.VMEM((_NP,), jnp.float32),
        pltpu.VMEM((_NP,), jnp.float32),
        pltpu.VMEM((_NP,), jnp.float32),
        pltpu.VMEM((_NP,), jnp.float32),
        pltpu.VMEM((_NP,), jnp.float32),
        pltpu.VMEM((_NP,), jnp.float32),
        pltpu.VMEM((_NP,), jnp.float32),
        pltpu.VMEM((_NP,), jnp.float32),
        pltpu.VMEM((_L,), jnp.float32),
        pltpu.VMEM((_L,), jnp.int32),
        pltpu.VMEM((_MAXOUT * 8,), jnp.float32),
    ],
)
def _nms_sc(x1h, y1h, x2h, y2h, sch, labh, outh,
            x1, y1, x2, y2, sc, labf, bx1, by1, bx2, by2, area,
            redv, redi, outbuf):
    cid = lax.axis_index("c")
    sid = lax.axis_index("s")
    wid = sid * 2 + cid

    @pl.when(wid < _B)
    def _():
        img = wid
        pltpu.sync_copy(x1h.at[img], x1)
        pltpu.sync_copy(y1h.at[img], y1)
        pltpu.sync_copy(x2h.at[img], x2)
        pltpu.sync_copy(y2h.at[img], y2)
        pltpu.sync_copy(sch.at[img], sc)
        pltpu.sync_copy(labh.at[img], labf)

        ninf = jnp.full((_L,), _NEG_INF, jnp.float32)
        zidx = jnp.zeros((_L,), jnp.int32)
        lane = lax.iota(jnp.int32, _L)

        def lane_max(v):
            for st in (1, 2, 4, 8):
                redv[...] = v
                gv = plsc.load_gather(redv, [lane ^ st])
                v = jnp.maximum(v, gv)
            return v

        def lane_argmax(v, i):
            for st in (1, 2, 4, 8):
                redv[...] = v
                redi[...] = i
                perm = lane ^ st
                gv = plsc.load_gather(redv, [perm])
                gi = plsc.load_gather(redi, [perm])
                take = (gv > v) | ((gv == v) & (gi < i))
                v = jnp.where(take, gv, v)
                i = jnp.where(take, gi, i)
            return v, i

        def mc_body(k, acc):
            for u in range(2):
                b = (2 * k + u) * _L
                acc = jnp.maximum(
                    jnp.maximum(acc, jnp.maximum(x1[pl.ds(b, _L)],
                                                 y1[pl.ds(b, _L)])),
                    jnp.maximum(x2[pl.ds(b, _L)], y2[pl.ds(b, _L)]))
            return acc

        mcv = lax.fori_loop(0, _NCH2, mc_body, ninf)
        mcp1 = lane_max(mcv) + jnp.float32(1.0)

        def stage_body(k, carry):
            bv, bi = carry
            for u in range(2):
                b = (2 * k + u) * _L
                l = labf[pl.ds(b, _L)] * mcp1
                a1 = x1[pl.ds(b, _L)] + l
                a2 = y1[pl.ds(b, _L)] + l
                a3 = x2[pl.ds(b, _L)] + l
                a4 = y2[pl.ds(b, _L)] + l
                bx1[pl.ds(b, _L)] = a1
                by1[pl.ds(b, _L)] = a2
                bx2[pl.ds(b, _L)] = a3
                by2[pl.ds(b, _L)] = a4
                area[pl.ds(b, _L)] = (a3 - a1) * (a4 - a2)
                s0 = sc[pl.ds(b, _L)]
                s0 = jnp.where(s0 >= _SCORE_THR, s0, _NEG_INF)
                sc[pl.ds(b, _L)] = s0
                idx = b + lane
                upd = s0 > bv
                bv = jnp.where(upd, s0, bv)
                bi = jnp.where(upd, idx, bi)
            return bv, bi

        bv0, bi0 = lax.fori_loop(0, _NCH2, stage_body, (ninf, zidx))

        def iter_body(t, carry):
            bv, bi = carry
            mv, iv = lane_argmax(bv, bi)
            validv = mv > _NEG_INF

            g1 = plsc.load_gather(x1, [iv])
            g2 = plsc.load_gather(y1, [iv])
            g3 = plsc.load_gather(x2, [iv])
            g4 = plsc.load_gather(y2, [iv])
            gl = plsc.load_gather(labf, [iv])
            row = jnp.where(lane == 0, g1,
                  jnp.where(lane == 1, g2,
                  jnp.where(lane == 2, g3,
                  jnp.where(lane == 3, g4,
                  jnp.where(lane == 4, mv,
                  jnp.where(lane == 5, gl, jnp.float32(0.0)))))))
            row = jnp.where(validv, row, jnp.float32(0.0))
            plsc.store_scatter(outbuf, [t * 8 + lane], row, mask=lane < 8)

            sx1 = plsc.load_gather(bx1, [iv])
            sy1 = plsc.load_gather(by1, [iv])
            sx2 = plsc.load_gather(bx2, [iv])
            sy2 = plsc.load_gather(by2, [iv])
            sar = plsc.load_gather(area, [iv])

            def sup_body(k, c2):
                nbv, nbi = c2
                for u in range(2):
                    b = (2 * k + u) * _L
                    c1 = bx1[pl.ds(b, _L)]
                    cc2 = by1[pl.ds(b, _L)]
                    c3 = bx2[pl.ds(b, _L)]
                    c4 = by2[pl.ds(b, _L)]
                    ar = area[pl.ds(b, _L)]
                    s0 = sc[pl.ds(b, _L)]
                    xx1 = jnp.maximum(sx1, c1)
                    yy1 = jnp.maximum(sy1, cc2)
                    xx2 = jnp.minimum(sx2, c3)
                    yy2 = jnp.minimum(sy2, c4)
                    w = jnp.maximum(xx2 - xx1, jnp.float32(0.0))
                    h = jnp.maximum(yy2 - yy1, jnp.float32(0.0))
                    inter = w * h
                    iou = inter / (sar + ar - inter + jnp.float32(1e-9))
                    idx = b + lane
                    s0 = jnp.where((iou > _IOU_THR) | (idx == iv),
                                   _NEG_INF, s0)
                    sc[pl.ds(b, _L)] = s0
                    upd = s0 > nbv
                    nbv = jnp.where(upd, s0, nbv)
                    nbi = jnp.where(upd, idx, nbi)
                return nbv, nbi

            return lax.fori_loop(0, _NCH2, sup_body, (ninf, zidx))

        lax.fori_loop(0, _MAXOUT, iter_body, (bv0, bi0))
        pltpu.sync_copy(outbuf, outh.at[img])


def kernel(boxes, scores, labels):
    pad = _NP - _N

    def padr(a, v):
        return jnp.pad(a, ((0, 0), (0, pad)), constant_values=v)

    x1p = padr(boxes[..., 0], 0.0)
    y1p = padr(boxes[..., 1], 0.0)
    x2p = padr(boxes[..., 2], 0.0)
    y2p = padr(boxes[..., 3], 0.0)
    scp = padr(scores, 0.0)
    labp = padr(labels.astype(jnp.float32), 0.0)
    out = _nms_sc(x1p, y1p, x2p, y2p, scp, labp)
    return out.reshape(_B, _MAXOUT, 8)[:, :, :6]
